# baseline (speedup 1.0000x reference)
import functools
import math

import jax
import jax.numpy as jnp
import numpy as np
from jax import lax
from jax.experimental import pallas as pl
from jax.experimental.pallas import tpu as pltpu
from jax.experimental.pallas import tpu_sc as plsc

_PE_LEN = 2048
_LANES = 16
_NC, _NS = 2, 16
_NW = _NC * _NS
_NB = 4
_AHEAD = 2
_TBLK = 8192


def _pos_encoding(length: int, depth: int) -> np.ndarray:
    pos = np.arange(length, dtype=np.float64)[:, None]
    i = np.arange(depth, dtype=np.float64)[None, :]
    exponent = (i - (i % 2)) / depth
    angle = pos / np.power(10000.0, exponent)
    pe = np.where((np.arange(depth)[None, :] % 2) == 0, np.sin(angle), np.cos(angle))
    return np.asarray(pe, dtype=np.float32)


_PE = _pos_encoding(_PE_LEN, 64)


@functools.cache
def _build_transpose(V: int, D: int):
    scale = np.float32(math.sqrt(D))

    def body(t_ref, o_ref):
        o_ref[...] = jnp.swapaxes(t_ref[...], 0, 1) * scale

    grid = (V + _TBLK - 1) // _TBLK
    return pl.pallas_call(
        body,
        grid=(grid,),
        in_specs=[pl.BlockSpec((D, _TBLK), lambda i: (0, i))],
        out_specs=pl.BlockSpec((_TBLK, D), lambda i: (i, 0)),
        out_shape=jax.ShapeDtypeStruct((V, D), jnp.float32),
    )


@functools.cache
def _build_gather(B: int, T: int, V: int, D: int):
    assert B % (128 * _NW) == 0 and D % _LANES == 0 and D % 8 == 0
    assert T % _NB == 0 and T > _NB + _AHEAD
    spw = B // _NW
    kt_n = D // 8
    kg_n = D // _LANES

    mesh = plsc.VectorSubcoreMesh(
        core_axis_name="c", subcore_axis_name="s",
        num_cores=_NC, num_subcores=_NS)

    scratch = [
        pltpu.VMEM((T, spw), jnp.int32),
        pltpu.VMEM((T, D), jnp.float32),
    ]
    scratch += [pltpu.VMEM((spw, D), jnp.float32) for _ in range(_NB)]
    scratch += [pltpu.VMEM((kt_n, 8, spw), jnp.float32) for _ in range(_NB)]
    scratch += [pltpu.SemaphoreType.DMA for _ in range(2 * _NB)]

    @functools.partial(
        pl.kernel,
        out_type=jax.ShapeDtypeStruct((T, kt_n, _NW, 8, spw), jnp.float32),
        mesh=mesh,
        scratch_types=scratch,
        compiler_params=pltpu.CompilerParams(
            use_tc_tiling_on_sc=False, needs_layout_passes=False),
    )
    def run(xt_hbm, pe_hbm, tab_hbm, out_hbm, *refs):
        idx_v = refs[0]
        pe_v = refs[1]
        rows = refs[2:2 + _NB]
        chunk = refs[2 + _NB:2 + 2 * _NB]
        gsem = refs[2 + 2 * _NB:2 + 3 * _NB]
        osem = refs[2 + 3 * _NB:2 + 4 * _NB]

        wid = lax.axis_index("s") * _NC + lax.axis_index("c")
        pltpu.sync_copy(pe_hbm, pe_v)
        pltpu.sync_copy(xt_hbm.at[:, pl.ds(wid * spw, spw)], idx_v)

        iota = lax.iota(jnp.int32, _LANES)
        kt_vecs = [lax.shift_right_logical(iota + kg * _LANES, 3)
                   for kg in range(kg_n)]
        k8_vecs = [jnp.bitwise_and(iota + kg * _LANES, 7)
                   for kg in range(kg_n)]

        def start_gather(t, slot):
            pltpu.async_copy(tab_hbm.at[idx_v.at[t]], rows[slot], gsem[slot])

        def drain_rows(slot):
            pltpu.make_async_copy(
                tab_hbm.at[pl.ds(0, spw)], rows[slot], gsem[slot]).wait()

        def drain_chunk(slot):
            for kt in range(kt_n):
                pltpu.make_async_copy(
                    out_hbm.at[0, 0, 0], chunk[slot].at[kt], osem[slot]).wait()

        for t0 in range(_AHEAD):
            start_gather(t0, t0)

        @pl.loop(0, T, step=_NB)
        def _round(s):
            for bb in range(_NB):
                t = s + bb
                j = t + _AHEAD
                slot_j = (bb + _AHEAD) % _NB

                @pl.when(j < T)
                def _():
                    @pl.when(j >= _NB)
                    def _():
                        drain_chunk(slot_j)
                    start_gather(j, slot_j)

                drain_rows(bb)

                pe_vecs = [pe_v[t, pl.ds(kg * _LANES, _LANES)]
                           for kg in range(kg_n)]

                @pl.loop(0, spw)
                def _row(b):
                    b_vec = jnp.full((_LANES,), b, jnp.int32)
                    for kg in range(kg_n):
                        val = rows[bb][b, pl.ds(kg * _LANES, _LANES)] \
                            + pe_vecs[kg]
                        plsc.store_scatter(
                            chunk[bb], [kt_vecs[kg], k8_vecs[kg], b_vec], val)

                for kt in range(kt_n):
                    pltpu.async_copy(chunk[bb].at[kt],
                                     out_hbm.at[t, kt, wid], osem[bb])

        for bb in range(_NB):
            drain_chunk(bb)

    return run


def kernel(x, table):
    B, T = x.shape
    V, D = table.shape
    xt = x.T.astype(jnp.int32)
    pe = jnp.asarray(_PE[:T, :D])
    tab_rm = _build_transpose(V, D)(table.T)
    out5 = _build_gather(B, T, V, D)(xt, pe, tab_rm)
    return out5.transpose(2, 4, 0, 1, 3).reshape(B, T, D)


# speedup vs baseline: 1.5136x; 1.5136x over previous
import functools
import math

import jax
import jax.numpy as jnp
import numpy as np
from jax import lax
from jax.experimental import pallas as pl
from jax.experimental.pallas import tpu as pltpu
from jax.experimental.pallas import tpu_sc as plsc

_PE_LEN = 2048
_LANES = 16
_NC, _NS = 2, 16
_NW = _NC * _NS
_NB = 4
_AHEAD = 2
_TBLK = 8192


def _pos_encoding(length: int, depth: int) -> np.ndarray:
    pos = np.arange(length, dtype=np.float64)[:, None]
    i = np.arange(depth, dtype=np.float64)[None, :]
    exponent = (i - (i % 2)) / depth
    angle = pos / np.power(10000.0, exponent)
    pe = np.where((np.arange(depth)[None, :] % 2) == 0, np.sin(angle), np.cos(angle))
    return np.asarray(pe, dtype=np.float32)


_PE = _pos_encoding(_PE_LEN, 64)


@functools.cache
def _build_transpose(V: int, D: int):
    scale = np.float32(math.sqrt(D))

    def body(t_ref, o_ref):
        o_ref[...] = jnp.swapaxes(t_ref[...], 0, 1) * scale

    grid = (V + _TBLK - 1) // _TBLK
    return pl.pallas_call(
        body,
        grid=(grid,),
        in_specs=[pl.BlockSpec((D, _TBLK), lambda i: (0, i))],
        out_specs=pl.BlockSpec((_TBLK, D), lambda i: (i, 0)),
        out_shape=jax.ShapeDtypeStruct((V, D), jnp.float32),
    )


@functools.cache
def _build_gather(B: int, T: int, V: int, D: int):
    assert B % (128 * _NW) == 0 and D % _LANES == 0 and D % 8 == 0
    assert T % _NB == 0 and T > _NB + _AHEAD
    spw = B // _NW
    kt_n = D // 8
    kg_n = D // _LANES

    mesh = plsc.VectorSubcoreMesh(
        core_axis_name="c", subcore_axis_name="s",
        num_cores=_NC, num_subcores=_NS)

    scratch = [
        pltpu.VMEM((T, spw), jnp.int32),
        pltpu.VMEM((T, D), jnp.float32),
    ]
    scratch += [pltpu.VMEM((spw, D), jnp.float32) for _ in range(_NB)]
    scratch += [pltpu.VMEM((kt_n, 8, spw + 1), jnp.float32) for _ in range(_NB)]
    scratch += [pltpu.SemaphoreType.DMA for _ in range(2 * _NB)]

    @functools.partial(
        pl.kernel,
        out_type=jax.ShapeDtypeStruct((T, kt_n, _NW, 8, spw), jnp.float32),
        mesh=mesh,
        scratch_types=scratch,
        compiler_params=pltpu.CompilerParams(
            use_tc_tiling_on_sc=False, needs_layout_passes=False),
    )
    def run(xt_hbm, pe_hbm, tab_hbm, out_hbm, *refs):
        idx_v = refs[0]
        pe_v = refs[1]
        rows = refs[2:2 + _NB]
        chunk = refs[2 + _NB:2 + 2 * _NB]
        gsem = refs[2 + 2 * _NB:2 + 3 * _NB]
        osem = refs[2 + 3 * _NB:2 + 4 * _NB]

        wid = lax.axis_index("s") * _NC + lax.axis_index("c")
        pltpu.sync_copy(pe_hbm, pe_v)
        pltpu.sync_copy(xt_hbm.at[:, pl.ds(wid * spw, spw)], idx_v)

        iota = lax.iota(jnp.int32, _LANES)
        kt_vecs = [lax.shift_right_logical(iota + kg * _LANES, 3)
                   for kg in range(kg_n)]
        k8_vecs = [jnp.bitwise_and(iota + kg * _LANES, 7)
                   for kg in range(kg_n)]

        def start_gather(t, slot):
            pltpu.async_copy(tab_hbm.at[idx_v.at[t]], rows[slot], gsem[slot])

        def drain_rows(slot):
            pltpu.make_async_copy(
                tab_hbm.at[pl.ds(0, spw)], rows[slot], gsem[slot]).wait()

        def drain_chunk(slot):
            for kt in range(kt_n):
                pltpu.make_async_copy(
                    out_hbm.at[0, 0, 0],
                    chunk[slot].at[kt, :, pl.ds(0, spw)], osem[slot]).wait()

        for t0 in range(_AHEAD):
            start_gather(t0, t0)

        @pl.loop(0, T, step=_NB)
        def _round(s):
            for bb in range(_NB):
                t = s + bb
                j = t + _AHEAD
                slot_j = (bb + _AHEAD) % _NB

                @pl.when(j < T)
                def _():
                    @pl.when(j >= _NB)
                    def _():
                        drain_chunk(slot_j)
                    start_gather(j, slot_j)

                drain_rows(bb)

                pe_vecs = [pe_v[t, pl.ds(kg * _LANES, _LANES)]
                           for kg in range(kg_n)]

                @pl.loop(0, spw)
                def _row(b):
                    b_vec = jnp.full((_LANES,), b, jnp.int32)
                    for kg in range(kg_n):
                        val = rows[bb][b, pl.ds(kg * _LANES, _LANES)] \
                            + pe_vecs[kg]
                        plsc.store_scatter(
                            chunk[bb], [kt_vecs[kg], k8_vecs[kg], b_vec], val)

                for kt in range(kt_n):
                    pltpu.async_copy(chunk[bb].at[kt, :, pl.ds(0, spw)],
                                     out_hbm.at[t, kt, wid], osem[bb])

        for bb in range(_NB):
            drain_chunk(bb)

    return run


def kernel(x, table):
    B, T = x.shape
    V, D = table.shape
    xt = x.T.astype(jnp.int32)
    pe = jnp.asarray(_PE[:T, :D])
    tab_rm = _build_transpose(V, D)(table.T)
    out5 = _build_gather(B, T, V, D)(xt, pe, tab_rm)
    return out5.transpose(2, 4, 0, 1, 3).reshape(B, T, D)


# speedup vs baseline: 1.5196x; 1.0039x over previous
import functools
import math

import jax
import jax.numpy as jnp
import numpy as np
from jax import lax
from jax.experimental import pallas as pl
from jax.experimental.pallas import tpu as pltpu
from jax.experimental.pallas import tpu_sc as plsc

_PE_LEN = 2048
_LANES = 16
_NC, _NS = 2, 16
_NW = _NC * _NS
_NB = 4
_AHEAD = 2
_TBLK = 8192


def _pos_encoding(length: int, depth: int) -> np.ndarray:
    pos = np.arange(length, dtype=np.float64)[:, None]
    i = np.arange(depth, dtype=np.float64)[None, :]
    exponent = (i - (i % 2)) / depth
    angle = pos / np.power(10000.0, exponent)
    pe = np.where((np.arange(depth)[None, :] % 2) == 0, np.sin(angle), np.cos(angle))
    return np.asarray(pe, dtype=np.float32)


_PE = _pos_encoding(_PE_LEN, 64)


@functools.cache
def _build_transpose(V: int, D: int):
    scale = np.float32(math.sqrt(D))
    eye = np.eye(D, dtype=np.float32) * scale

    def body(t_ref, e_ref, o_ref):
        o_ref[...] = lax.dot_general(
            t_ref[...], e_ref[...],
            dimension_numbers=(((0,), (0,)), ((), ())),
            preferred_element_type=jnp.float32)

    grid = (V + _TBLK - 1) // _TBLK
    call = pl.pallas_call(
        body,
        grid=(grid,),
        in_specs=[pl.BlockSpec((D, _TBLK), lambda i: (0, i)),
                  pl.BlockSpec((D, D), lambda i: (0, 0))],
        out_specs=pl.BlockSpec((_TBLK, D), lambda i: (i, 0)),
        out_shape=jax.ShapeDtypeStruct((V, D), jnp.float32),
    )
    return lambda tab_t: call(tab_t, jnp.asarray(eye))


@functools.cache
def _build_gather(B: int, T: int, V: int, D: int):
    assert B % (128 * _NW) == 0 and D % _LANES == 0 and D % 8 == 0
    assert T % _NB == 0 and T > _NB + _AHEAD
    spw = B // _NW
    kt_n = D // 8
    kg_n = D // _LANES

    mesh = plsc.VectorSubcoreMesh(
        core_axis_name="c", subcore_axis_name="s",
        num_cores=_NC, num_subcores=_NS)

    scratch = [
        pltpu.VMEM((T, spw), jnp.int32),
        pltpu.VMEM((T, D), jnp.float32),
    ]
    scratch += [pltpu.VMEM((spw, D), jnp.float32) for _ in range(_NB)]
    scratch += [pltpu.VMEM((D, spw + 1), jnp.float32) for _ in range(_NB)]
    scratch += [pltpu.SemaphoreType.DMA for _ in range(2 * _NB)]

    @functools.partial(
        pl.kernel,
        out_type=jax.ShapeDtypeStruct((T, kt_n, _NW, 8, spw), jnp.float32),
        mesh=mesh,
        scratch_types=scratch,
        compiler_params=pltpu.CompilerParams(
            use_tc_tiling_on_sc=False, needs_layout_passes=False),
    )
    def run(xt_hbm, pe_hbm, tab_hbm, out_hbm, *refs):
        idx_v = refs[0]
        pe_v = refs[1]
        rows = refs[2:2 + _NB]
        chunk = refs[2 + _NB:2 + 2 * _NB]
        gsem = refs[2 + 2 * _NB:2 + 3 * _NB]
        osem = refs[2 + 3 * _NB:2 + 4 * _NB]

        wid = lax.axis_index("s") * _NC + lax.axis_index("c")
        pltpu.sync_copy(pe_hbm, pe_v)
        pltpu.sync_copy(xt_hbm.at[:, pl.ds(wid * spw, spw)], idx_v)

        iota = lax.iota(jnp.int32, _LANES)
        k_vecs = [iota + kg * _LANES for kg in range(kg_n)]

        def start_gather(t, slot):
            pltpu.async_copy(tab_hbm.at[idx_v.at[t]], rows[slot], gsem[slot])

        def drain_rows(slot):
            pltpu.make_async_copy(
                tab_hbm.at[pl.ds(0, spw)], rows[slot], gsem[slot]).wait()

        def drain_chunk(slot):
            for kt in range(kt_n):
                pltpu.make_async_copy(
                    out_hbm.at[0, 0, 0],
                    chunk[slot].at[pl.ds(kt * 8, 8), pl.ds(0, spw)],
                    osem[slot]).wait()

        for t0 in range(_AHEAD):
            start_gather(t0, t0)

        @pl.loop(0, T, step=_NB)
        def _round(s):
            for bb in range(_NB):
                t = s + bb
                j = t + _AHEAD
                slot_j = (bb + _AHEAD) % _NB

                @pl.when(j < T)
                def _():
                    @pl.when(j >= _NB)
                    def _():
                        drain_chunk(slot_j)
                    start_gather(j, slot_j)

                drain_rows(bb)

                pe_vecs = [pe_v[t, pl.ds(kg * _LANES, _LANES)]
                           for kg in range(kg_n)]

                @pl.loop(0, spw, unroll=8)
                def _row(b):
                    b_vec = jnp.full((_LANES,), b, jnp.int32)
                    for kg in range(kg_n):
                        val = rows[bb][b, pl.ds(kg * _LANES, _LANES)] \
                            + pe_vecs[kg]
                        plsc.store_scatter(
                            chunk[bb], [k_vecs[kg], b_vec], val)

                for kt in range(kt_n):
                    pltpu.async_copy(
                        chunk[bb].at[pl.ds(kt * 8, 8), pl.ds(0, spw)],
                        out_hbm.at[t, kt, wid], osem[bb])

        for bb in range(_NB):
            drain_chunk(bb)

    return run


def kernel(x, table):
    B, T = x.shape
    V, D = table.shape
    xt = x.T.astype(jnp.int32)
    pe = jnp.asarray(_PE[:T, :D])
    tab_rm = _build_transpose(V, D)(table.T)
    out5 = _build_gather(B, T, V, D)(xt, pe, tab_rm)
    return out5.transpose(2, 4, 0, 1, 3).reshape(B, T, D)
